# baseline (speedup 1.0000x reference)
import functools

import jax
import jax.numpy as jnp
import numpy as np
from jax import lax
from jax.experimental import pallas as pl
from jax.experimental.pallas import tpu as pltpu
from jax.experimental.pallas import tpu_sc as plsc

N = 10000
NP = 10240
TRASH = N
E = 320000
CHUNK = 128
NTILES = 16
NCHUNKS = 157
EPT = CHUNK * NCHUNKS
EP = EPT * NTILES
ROWS_PT = NP // NTILES
D = 128
H = 256
BN_SCALE = float(1.0 / np.sqrt(1.0 + 1e-5))
_PREC = lax.Precision.HIGHEST


def _make_sc_agg(Fh, with_count):
    mesh = plsc.VectorSubcoreMesh(core_axis_name="core", subcore_axis_name="subcore")
    out_type = [
        jax.ShapeDtypeStruct((NP, Fh), jnp.float32),
        jax.ShapeDtypeStruct((NP, Fh), jnp.float32),
    ]
    scratch = [
        pltpu.VMEM((CHUNK,), jnp.int32),
        pltpu.VMEM((CHUNK,), jnp.int32),
        pltpu.VMEM((CHUNK,), jnp.int32),
        pltpu.VMEM((CHUNK,), jnp.int32),
        pltpu.VMEM((CHUNK,), jnp.int32),
        pltpu.VMEM((CHUNK,), jnp.int32),
        pltpu.VMEM((CHUNK, Fh), jnp.float32),
        pltpu.VMEM((CHUNK, Fh), jnp.float32),
        pltpu.VMEM_SHARED((NP, Fh), jnp.float32),
        pltpu.SemaphoreType.DMA,
        pltpu.SemaphoreType.DMA,
        pltpu.SemaphoreType.DMA,
        pltpu.SemaphoreType.DMA,
        pltpu.SemaphoreType.DMA,
        pltpu.SemaphoreType.DMA,
    ]
    if with_count:
        out_type.append(jax.ShapeDtypeStruct((NP, 16), jnp.float32))
        scratch += [
            pltpu.VMEM((CHUNK, 16), jnp.float32),
            pltpu.VMEM_SHARED((NP, 16), jnp.float32),
        ]

    @functools.partial(pl.kernel, out_type=out_type, mesh=mesh,
                       scratch_types=scratch,
                       compiler_params=pltpu.CompilerParams(
                           use_tc_tiling_on_sc=False))
    def body(hlo, hhi, src, dst, out_lo, out_hi, *rest):
        if with_count:
            (cnt_out, sidxA, didxA, sidxB, didxB, sdidxA, sdidxB,
             rowsA, rowsB, acc, semA, semB, semGA, semGB,
             semSA, semSB, ones, cacc) = rest
        else:
            (sidxA, didxA, sidxB, didxB, sdidxA, sdidxB,
             rowsA, rowsB, acc, semA, semB, semGA, semGB,
             semSA, semSB) = rest
        c = lax.axis_index("core")
        t = lax.axis_index("subcore")

        @pl.loop(0, CHUNK)
        def _(r):
            @pl.loop(0, Fh // 16)
            def _(j):
                rowsA.at[r, pl.ds(j * 16, 16)][...] = jnp.zeros((16,), jnp.float32)
            if with_count:
                ones.at[r, pl.ds(0, 16)][...] = jnp.zeros((16,), jnp.float32)

        @pl.loop(0, ROWS_PT // CHUNK)
        def _(j):
            pltpu.sync_copy(rowsA, acc.at[pl.ds(t * ROWS_PT + j * CHUNK, CHUNK)])
            if with_count:
                @pl.when(c == 0)
                def _():
                    pltpu.sync_copy(ones, cacc.at[pl.ds(t * ROWS_PT + j * CHUNK, CHUNK)])

        if with_count:
            @pl.loop(0, CHUNK)
            def _(r):
                ones.at[r, pl.ds(0, 16)][...] = jnp.ones((16,), jnp.float32)

        plsc.subcore_barrier()

        base = t * EPT
        half = (NCHUNKS - 1) // 2

        def fetch(buf_s, buf_d, off, sem):
            pltpu.async_copy(src.at[pl.ds(off, CHUNK)], buf_s, sem)
            pltpu.async_copy(dst.at[pl.ds(off, CHUNK)], buf_d, sem)

        def wait_fetch(buf_s, buf_d, off, sem):
            pltpu.make_async_copy(src.at[pl.ds(off, CHUNK)], buf_s, sem).wait()
            pltpu.make_async_copy(dst.at[pl.ds(off, CHUNK)], buf_d, sem).wait()

        A = (sidxA, didxA, sdidxA, rowsA, semA, semGA, semSA)
        B = (sidxB, didxB, sdidxB, rowsB, semB, semGB, semSB)

        def gather_start(X):
            sidx_, _, _, rows_, _, semG_, _ = X

            @pl.when(c == 0)
            def _():
                pltpu.async_copy(hlo.at[sidx_], rows_, semG_)

            @pl.when(c == 1)
            def _():
                pltpu.async_copy(hhi.at[sidx_], rows_, semG_)

        def gather_wait(X):
            sidx_, _, _, rows_, _, semG_, _ = X

            @pl.when(c == 0)
            def _():
                pltpu.make_async_copy(hlo.at[sidx_], rows_, semG_).wait()

            @pl.when(c == 1)
            def _():
                pltpu.make_async_copy(hhi.at[sidx_], rows_, semG_).wait()

        def scatter_start(X):
            _, didx_, sdidx_, rows_, _, _, semS_ = X
            for j in range(CHUNK // 16):
                sdidx_.at[pl.ds(j * 16, 16)][...] = didx_.at[pl.ds(j * 16, 16)][...]
            pltpu.async_copy(rows_, acc.at[sdidx_], semS_, add=True)
            if with_count:
                @pl.when(c == 0)
                def _():
                    pltpu.sync_copy(ones, cacc.at[sdidx_], add=True)

        def scatter_wait(X):
            _, _, sdidx_, rows_, _, _, semS_ = X
            pltpu.make_async_copy(rows_, acc.at[sdidx_], semS_).wait()

        def idx_fetch(X, off):
            sidx_, didx_, _, _, semi_, _, _ = X
            fetch(sidx_, didx_, off, semi_)

        def idx_wait(X, off):
            sidx_, didx_, _, _, semi_, _, _ = X
            wait_fetch(sidx_, didx_, off, semi_)

        idx_fetch(A, base)
        idx_fetch(B, base + CHUNK)
        idx_wait(A, base)
        gather_start(A)

        @pl.loop(0, half)
        def _(k):
            offa = base + (2 * k) * CHUNK
            gather_wait(A)
            scatter_start(A)
            idx_fetch(A, offa + 2 * CHUNK)
            idx_wait(B, offa + CHUNK)

            @pl.when(k > 0)
            def _():
                scatter_wait(B)

            gather_start(B)

            gather_wait(B)
            scatter_start(B)

            @pl.when(k < half - 1)
            def _():
                idx_fetch(B, offa + 3 * CHUNK)

            idx_wait(A, offa + 2 * CHUNK)
            scatter_wait(A)
            gather_start(A)

        gather_wait(A)
        scatter_start(A)
        scatter_wait(B)
        scatter_wait(A)

        plsc.subcore_barrier()

        rs = pl.ds(t * ROWS_PT, ROWS_PT)

        @pl.when(c == 0)
        def _():
            pltpu.sync_copy(acc.at[rs], out_lo.at[rs])
            if with_count:
                pltpu.sync_copy(cacc.at[rs], cnt_out.at[rs])

        @pl.when(c == 1)
        def _():
            pltpu.sync_copy(acc.at[rs], out_hi.at[rs])

    return body


_sc_agg0 = _make_sc_agg(D // 2, with_count=True)
_sc_agg = _make_sc_agg(H // 2, with_count=False)


_R = 2048


def _dot_t(a, w):
    return lax.dot_general(a, w, (((1,), (1,)), ((), ())),
                           precision=_PREC, preferred_element_type=jnp.float32)


def _layer_body(agl, agh, cnt, hl, hh, wl, bl, wr, g, be, olo, ohi):
    inv = 1.0 / jnp.maximum(cnt[...][:, 0:1], 1.0)
    agg = jnp.concatenate([agl[...], agh[...]], axis=1) * inv
    h = jnp.concatenate([hl[...], hh[...]], axis=1)
    z = _dot_t(agg, wl[...]) + _dot_t(h, wr[...]) + bl[...]
    hn = jnp.maximum(g[...] * (z * BN_SCALE) + be[...], 0.0)
    olo[...] = hn[:, : H // 2]
    ohi[...] = hn[:, H // 2:]


def _head_body(agl, agh, cnt, hl, hh, wl, bl, wr, g, be, wc1, bc1, wc2, bc2, out):
    inv = 1.0 / jnp.maximum(cnt[...][:, 0:1], 1.0)
    agg = jnp.concatenate([agl[...], agh[...]], axis=1) * inv
    h = jnp.concatenate([hl[...], hh[...]], axis=1)
    z = _dot_t(agg, wl[...]) + _dot_t(h, wr[...]) + bl[...]
    h3 = jnp.maximum(g[...] * (z * BN_SCALE) + be[...], 0.0)
    t1 = jnp.maximum(_dot_t(h3, wc1[...]) + bc1[...], 0.0)
    logits = _dot_t(t1, wc2[...]) + bc2[...]
    m = jnp.max(logits, axis=1, keepdims=True)
    lse = m + jnp.log(jnp.sum(jnp.exp(logits - m), axis=1, keepdims=True))
    out[...] = logits - lse


def _row_spec(fw):
    return pl.BlockSpec((_R, fw), lambda i: (i, 0))


def _full_spec(shape):
    return pl.BlockSpec(shape, lambda i: tuple(0 for _ in shape))


def _make_tc_layer(Fin):
    Fh = Fin // 2
    in_specs = [
        _row_spec(Fh), _row_spec(Fh), _row_spec(16),
        _row_spec(Fh), _row_spec(Fh),
        _full_spec((H, Fin)), _full_spec((1, H)),
        _full_spec((H, Fin)),
        _full_spec((1, H)), _full_spec((1, H)),
    ]
    return pl.pallas_call(
        _layer_body,
        grid=(NP // _R,),
        in_specs=in_specs,
        out_specs=[_row_spec(H // 2), _row_spec(H // 2)],
        out_shape=[jax.ShapeDtypeStruct((NP, H // 2), jnp.float32)] * 2,
    )


_tc_layer0 = _make_tc_layer(D)
_tc_layer = _make_tc_layer(H)

_tc_head = pl.pallas_call(
    _head_body,
    grid=(NP // _R,),
    in_specs=[
        _row_spec(H // 2), _row_spec(H // 2), _row_spec(16),
        _row_spec(H // 2), _row_spec(H // 2),
        _full_spec((H, H)), _full_spec((1, H)),
        _full_spec((H, H)),
        _full_spec((1, H)), _full_spec((1, H)),
        _full_spec((H // 2, H)), _full_spec((1, H // 2)),
        _full_spec((2, H // 2)), _full_spec((1, 2)),
    ],
    out_specs=[_row_spec(2)],
    out_shape=[jax.ShapeDtypeStruct((NP, 2), jnp.float32)],
)


def kernel(x, edge_index, Wl0, bl0, Wr0, g0, be0, Wl1, bl1, Wr1, g1, be1,
           Wl2, bl2, Wr2, g2, be2, Wc1, bc1, Wc2, bc2):
    src = edge_index[0]
    dst = edge_index[1]
    pad = EP - E
    src_p = jnp.concatenate([src, jnp.zeros((pad,), jnp.int32)])
    dst_p = jnp.concatenate([dst, jnp.full((pad,), TRASH, jnp.int32)])

    xp = jnp.pad(x, ((0, NP - N), (0, 0)))
    x_lo = xp[:, : D // 2]
    x_hi = xp[:, D // 2:]

    r = lambda v: v.reshape(1, -1)

    agg0_lo, agg0_hi, cnt = _sc_agg0(x_lo, x_hi, src_p, dst_p)
    h1_lo, h1_hi = _tc_layer0(agg0_lo, agg0_hi, cnt, x_lo, x_hi,
                              Wl0, r(bl0), Wr0, r(g0), r(be0))
    agg1_lo, agg1_hi = _sc_agg(h1_lo, h1_hi, src_p, dst_p)
    h2_lo, h2_hi = _tc_layer(agg1_lo, agg1_hi, cnt, h1_lo, h1_hi,
                             Wl1, r(bl1), Wr1, r(g1), r(be1))
    agg2_lo, agg2_hi = _sc_agg(h2_lo, h2_hi, src_p, dst_p)
    (out_p,) = _tc_head(agg2_lo, agg2_hi, cnt, h2_lo, h2_hi,
                        Wl2, r(bl2), Wr2, r(g2), r(be2),
                        Wc1, r(bc1), Wc2, r(bc2))
    return out_p[:N]
